# baseline (speedup 1.0000x reference)
import jax
import jax.numpy as jnp
from jax import lax
from jax.experimental import pallas as pl
from jax.experimental.pallas import tpu as pltpu

B, S, D = 64, 577, 1024
E, R, H = 8, 64, 256
TOP_K = 2
CHUNK = 128

_MM = dict(dimension_numbers=(((1,), (0,)), ((), ())),
           preferred_element_type=jnp.float32)


def _body(x_ref, W1t_ref, b1_ref, W2t_ref, b2_ref, Wmt_ref, Ut_ref, S_ref,
          Vt_ref, bias_ref, out_ref):
    xb = x_ref[0]

    ones_row = jnp.full((1, S), 1.0 / S, dtype=jnp.float32)
    pooled = lax.dot_general(ones_row, xb, **_MM)
    h = lax.dot_general(pooled, W1t_ref[...], **_MM) + b1_ref[...]
    h = jnp.maximum(h, 0.0)
    logits = lax.dot_general(h, W2t_ref[...], **_MM) + b2_ref[...]

    iot = lax.broadcasted_iota(jnp.int32, (1, E), 1)
    m0 = jnp.max(logits)
    idx0 = jnp.min(jnp.where(logits == m0, iot, E))
    masked = jnp.where(iot == idx0, jnp.finfo(jnp.float32).min, logits)
    m1 = jnp.max(masked)
    idx1 = jnp.min(jnp.where(masked == m1, iot, E))
    e1 = jnp.exp(m1 - m0)
    g0 = 1.0 / (1.0 + e1)
    g1 = e1 * g0

    vcat = jnp.concatenate([Vt_ref[idx0], Vt_ref[idx1]], axis=1)
    ucat = jnp.concatenate([Ut_ref[idx0], Ut_ref[idx1]], axis=0)
    scat = jnp.concatenate([S_ref[idx0] * g0, S_ref[idx1] * g1],
                           axis=1)
    bias_row = bias_ref[...]

    for lo in range(0, S, CHUNK):
        n = min(CHUNK, S - lo)
        a = x_ref[0, pl.ds(lo, n)].astype(jnp.bfloat16)
        xv = lax.dot_general(a, vcat, **_MM)
        xv = (xv * scat).astype(jnp.bfloat16)
        acc = lax.dot_general(a, Wmt_ref[...], **_MM)
        acc = acc + lax.dot_general(xv, ucat, **_MM)
        out_ref[0, pl.ds(lo, n)] = acc + bias_row


@jax.jit
def kernel(x, W1, b1, W2, b2, weight_main, U_all, S_all, V_all, bias):
    W1t = W1.T
    W2t = W2.T
    Wmt = weight_main.T.astype(jnp.bfloat16)
    Ut_all = U_all.transpose(0, 2, 1).astype(jnp.bfloat16)
    Vt_all = V_all.transpose(0, 2, 1).astype(jnp.bfloat16)
    b1_2d = b1.reshape(1, H)
    b2_2d = b2.reshape(1, E)
    S_3d = S_all.reshape(E, 1, R)
    bias_2d = bias.reshape(1, D)

    grid = (B,)
    full = lambda shape: pl.BlockSpec(shape, lambda b: (0,) * len(shape))
    out = pl.pallas_call(
        _body,
        grid=grid,
        in_specs=[
            pl.BlockSpec((1, S, D), lambda b: (b, 0, 0)),
            full((D, H)),
            full((1, H)),
            full((H, E)),
            full((1, E)),
            full((D, D)),
            full((E, R, D)),
            full((E, 1, R)),
            full((E, D, R)),
            full((1, D)),
        ],
        out_specs=pl.BlockSpec((1, S, D), lambda b: (b, 0, 0)),
        out_shape=jax.ShapeDtypeStruct((B, S, D), jnp.float32),
        compiler_params=pltpu.CompilerParams(
            dimension_semantics=("parallel",),
        ),
    )(x, W1t, b1_2d, W2t, b2_2d, Wmt, Ut_all, S_3d, Vt_all, bias_2d)
    return out


# speedup vs baseline: 1.1756x; 1.1756x over previous
import jax
import jax.numpy as jnp
from jax import lax
from jax.experimental import pallas as pl
from jax.experimental.pallas import tpu as pltpu

B, S, D = 64, 577, 1024
E, R, H = 8, 64, 256
TOP_K = 2
IMGS = 2

_CT1 = dict(dimension_numbers=(((1,), (1,)), ((), ())),
            preferred_element_type=jnp.float32)


def _one_image(x_ref, W1_ref, b1_ref, W2_ref, b2_ref, Wm_ref, U_ref, S_ref,
               V_ref, bias_ref, out_ref, i):
    xb = x_ref[i]

    pooled = jnp.mean(xb, axis=0, keepdims=True)
    h = lax.dot_general(pooled, W1_ref[...], **_CT1) + b1_ref[...]
    h = jnp.maximum(h, 0.0)
    logits = lax.dot_general(h, W2_ref[...], **_CT1) + b2_ref[...]

    iot = lax.broadcasted_iota(jnp.int32, (1, E), 1)
    m0 = jnp.max(logits)
    idx0 = jnp.min(jnp.where(logits == m0, iot, E))
    masked = jnp.where(iot == idx0, jnp.finfo(jnp.float32).min, logits)
    m1 = jnp.max(masked)
    idx1 = jnp.min(jnp.where(masked == m1, iot, E))
    e1 = jnp.exp(m1 - m0)
    g0 = 1.0 / (1.0 + e1)
    g1 = e1 * g0

    vcat = jnp.concatenate([V_ref[idx0], V_ref[idx1]], axis=0)
    ucat = jnp.concatenate([U_ref[idx0], U_ref[idx1]], axis=1)
    scat = jnp.concatenate([S_ref[idx0] * g0, S_ref[idx1] * g1],
                           axis=1)

    xv = lax.dot_general(xb, vcat, **_CT1)
    xv = xv * scat
    expert = lax.dot_general(xv, ucat, **_CT1)
    main = lax.dot_general(xb, Wm_ref[...], **_CT1)

    out_ref[i] = main + expert + bias_ref[...]


def _body(*refs):
    for i in range(IMGS):
        _one_image(*refs, i)


@jax.jit
def kernel(x, W1, b1, W2, b2, weight_main, U_all, S_all, V_all, bias):
    b1_2d = b1.reshape(1, H)
    b2_2d = b2.reshape(1, E)
    S_3d = S_all.reshape(E, 1, R)
    bias_2d = bias.reshape(1, D)

    grid = (B // IMGS,)
    full = lambda shape: pl.BlockSpec(shape, lambda b: (0,) * len(shape))
    out = pl.pallas_call(
        _body,
        grid=grid,
        in_specs=[
            pl.BlockSpec((IMGS, S, D), lambda b: (b, 0, 0)),
            full((H, D)),
            full((1, H)),
            full((E, H)),
            full((1, E)),
            full((D, D)),
            full((E, D, R)),
            full((E, 1, R)),
            full((E, R, D)),
            full((1, D)),
        ],
        out_specs=pl.BlockSpec((IMGS, S, D), lambda b: (b, 0, 0)),
        out_shape=jax.ShapeDtypeStruct((B, S, D), jnp.float32),
        compiler_params=pltpu.CompilerParams(
            dimension_semantics=("arbitrary",),
        ),
    )(x, W1, b1_2d, W2, b2_2d, weight_main, U_all, S_3d, V_all, bias_2d)
    return out


# speedup vs baseline: 1.2374x; 1.0525x over previous
import jax
import jax.numpy as jnp
from jax import lax
from jax.experimental import pallas as pl
from jax.experimental.pallas import tpu as pltpu

B, S, D = 64, 577, 1024
E, R, H = 8, 64, 256
TOP_K = 2
IMGS = 4

_CT1 = dict(dimension_numbers=(((1,), (1,)), ((), ())),
            preferred_element_type=jnp.float32)


def _one_image(x_ref, W1_ref, b1_ref, W2_ref, b2_ref, Wm_ref, U_ref, S_ref,
               V_ref, bias_ref, out_ref, i):
    xb = x_ref[i]

    pooled = jnp.mean(xb, axis=0, keepdims=True)
    h = lax.dot_general(pooled, W1_ref[...], **_CT1) + b1_ref[...]
    h = jnp.maximum(h, 0.0)
    logits = lax.dot_general(h, W2_ref[...], **_CT1) + b2_ref[...]

    iot = lax.broadcasted_iota(jnp.int32, (1, E), 1)
    m0 = jnp.max(logits)
    idx0 = jnp.min(jnp.where(logits == m0, iot, E))
    masked = jnp.where(iot == idx0, jnp.finfo(jnp.float32).min, logits)
    m1 = jnp.max(masked)
    idx1 = jnp.min(jnp.where(masked == m1, iot, E))
    e1 = jnp.exp(m1 - m0)
    g0 = 1.0 / (1.0 + e1)
    g1 = e1 * g0

    vcat = jnp.concatenate([V_ref[idx0], V_ref[idx1]], axis=0)
    ucat = jnp.concatenate([U_ref[idx0], U_ref[idx1]], axis=1)
    scat = jnp.concatenate([S_ref[idx0] * g0, S_ref[idx1] * g1],
                           axis=1)

    xv = lax.dot_general(xb, vcat, **_CT1)
    xv = xv * scat
    expert = lax.dot_general(xv, ucat, **_CT1)
    main = lax.dot_general(xb, Wm_ref[...], **_CT1)

    out_ref[i] = main + expert + bias_ref[...]


def _body(*refs):
    for i in range(IMGS):
        _one_image(*refs, i)


@jax.jit
def kernel(x, W1, b1, W2, b2, weight_main, U_all, S_all, V_all, bias):
    b1_2d = b1.reshape(1, H)
    b2_2d = b2.reshape(1, E)
    S_3d = S_all.reshape(E, 1, R)
    bias_2d = bias.reshape(1, D)

    grid = (B // IMGS,)
    full = lambda shape: pl.BlockSpec(shape, lambda b: (0,) * len(shape))
    out = pl.pallas_call(
        _body,
        grid=grid,
        in_specs=[
            pl.BlockSpec((IMGS, S, D), lambda b: (b, 0, 0)),
            full((H, D)),
            full((1, H)),
            full((E, H)),
            full((1, E)),
            full((D, D)),
            full((E, D, R)),
            full((E, 1, R)),
            full((E, R, D)),
            full((1, D)),
        ],
        out_specs=pl.BlockSpec((IMGS, S, D), lambda b: (b, 0, 0)),
        out_shape=jax.ShapeDtypeStruct((B, S, D), jnp.float32),
        compiler_params=pltpu.CompilerParams(
            dimension_semantics=("arbitrary",),
        ),
    )(x, W1, b1_2d, W2, b2_2d, weight_main, U_all, S_3d, V_all, bias_2d)
    return out
